# Initial kernel scaffold; baseline (speedup 1.0000x reference)
import jax
import jax.numpy as jnp
from jax.experimental import pallas as pl


def kernel(data, x, adj, x_t, adj_t, clustering, W1, b1, W2, b2, W3, b3, W4, b4, gn1_w, gn1_b, gn1_a, gn2_w, gn2_b, gn2_a, gn3_w, gn3_b, gn3_a, gn4_w, gn4_b, gn4_a, bn1_g, bn1_b, bn2_g, bn2_b):
    raise NotImplementedError("write your pallas kernel here")


# speedup vs baseline: 6.9143x; 6.9143x over previous
import functools

import jax
import jax.numpy as jnp
from jax import lax
from jax.experimental import pallas as pl
from jax.experimental.pallas import tpu as pltpu
from jax.experimental.pallas import tpu_sc as plsc

N_CELLS = 10000
N_GENES = 2000
H = 128
E_CELL = 320000
E_GENE = 64000

NC = 2
NS = 16
NW = NC * NS
CHUNK = 128

KC = E_CELL // (NW * CHUNK) + (1 if E_CELL % (NW * CHUNK) else 0)
KC += KC % 2
KG = E_GENE // (NW * CHUNK) + (1 if E_GENE % (NW * CHUNK) else 0)

ZR_C = 632
ZR_G = 128
ACC_C = NS * ZR_C
ACC_G = NS * ZR_G

_EPS = 1e-5

_SC_MESH = plsc.VectorSubcoreMesh(core_axis_name="c", subcore_axis_name="s")


def _prep_edges(adj, n_nodes, k_chunks):
    e = adj.shape[0]
    pad = NW * k_chunks * CHUNK - e
    src = jnp.concatenate([adj[:, 0], jnp.zeros((pad,), jnp.int32)])
    dst = jnp.concatenate([adj[:, 1], jnp.full((pad,), n_nodes, jnp.int32)])
    return (src.reshape(NW, k_chunks, CHUNK), dst.reshape(NW, k_chunks, CHUNK))


@functools.partial(
    pl.kernel,
    out_type=(
        jax.ShapeDtypeStruct((NC, ACC_C, H), jnp.float32),
        jax.ShapeDtypeStruct((NC, ACC_G, H), jnp.float32),
    ),
    mesh=_SC_MESH,
    scratch_types=[
        pltpu.VMEM((KC, CHUNK), jnp.int32),
        pltpu.VMEM((KG, CHUNK), jnp.int32),
        pltpu.VMEM((CHUNK, H), jnp.float32),
        pltpu.VMEM_SHARED((ACC_C, H), jnp.float32),
        pltpu.VMEM_SHARED((ACC_G, H), jnp.float32),
    ],
)
def _sc_hist(dstc_hbm, dstg_hbm, ones_hbm, zc_hbm, zg_hbm, outc_hbm, outg_hbm,
             dstc_v, dstg_v, ones_v, degc_sh, degg_sh):
    cid = lax.axis_index("c")
    sid = lax.axis_index("s")
    wid = cid * NS + sid
    pltpu.sync_copy(dstc_hbm.at[wid], dstc_v)
    pltpu.sync_copy(dstg_hbm.at[wid], dstg_v)
    pltpu.sync_copy(ones_hbm, ones_v)
    pltpu.sync_copy(zc_hbm, degc_sh.at[pl.ds(sid * ZR_C, ZR_C)])
    pltpu.sync_copy(zg_hbm, degg_sh.at[pl.ds(sid * ZR_G, ZR_G)])
    plsc.subcore_barrier()

    def cbody(j, carry):
        pltpu.sync_copy(ones_v, degc_sh.at[dstc_v.at[j]], add=True)
        return carry

    lax.fori_loop(0, KC, cbody, 0)

    def gbody(j, carry):
        pltpu.sync_copy(ones_v, degg_sh.at[dstg_v.at[j]], add=True)
        return carry

    lax.fori_loop(0, KG, gbody, 0)
    plsc.subcore_barrier()
    pltpu.sync_copy(degc_sh.at[pl.ds(sid * ZR_C, ZR_C)],
                    outc_hbm.at[cid, pl.ds(sid * ZR_C, ZR_C)])
    pltpu.sync_copy(degg_sh.at[pl.ds(sid * ZR_G, ZR_G)],
                    outg_hbm.at[cid, pl.ds(sid * ZR_G, ZR_G)])


def _make_scatter(acc_rows, zr, k_chunks):
    @functools.partial(
        pl.kernel,
        out_type=jax.ShapeDtypeStruct((NC, acc_rows, H), jnp.float32),
        mesh=_SC_MESH,
        scratch_types=[
            pltpu.VMEM((k_chunks, CHUNK), jnp.int32),
            pltpu.VMEM((k_chunks, CHUNK), jnp.int32),
            pltpu.VMEM((CHUNK, H), jnp.float32),
            pltpu.VMEM_SHARED((acc_rows, H), jnp.float32),
        ],
    )
    def _scatter(g_hbm, src_hbm, dst_hbm, z_hbm, out_hbm,
                 src_v, dst_v, rows_v, acc_sh):
        cid = lax.axis_index("c")
        sid = lax.axis_index("s")
        wid = cid * NS + sid
        pltpu.sync_copy(src_hbm.at[wid], src_v)
        pltpu.sync_copy(dst_hbm.at[wid], dst_v)
        pltpu.sync_copy(z_hbm, acc_sh.at[pl.ds(sid * zr, zr)])
        plsc.subcore_barrier()

        def body(j, carry):
            pltpu.sync_copy(g_hbm.at[src_v.at[j]], rows_v)
            pltpu.sync_copy(rows_v, acc_sh.at[dst_v.at[j]], add=True)
            return carry

        lax.fori_loop(0, k_chunks, body, 0)
        plsc.subcore_barrier()
        pltpu.sync_copy(acc_sh.at[pl.ds(sid * zr, zr)],
                        out_hbm.at[cid, pl.ds(sid * zr, zr)])

    return _scatter


_sc_scatter_c = _make_scatter(ACC_C, ZR_C, KC)
_sc_scatter_g = _make_scatter(ACC_G, ZR_G, KG)


def _dis_body(dc_ref, dg_ref, oc_ref, og_ref):
    dc = dc_ref[0] + dc_ref[1]
    oc_ref[...] = lax.rsqrt(dc[:N_CELLS, 0:1] + 1.0)
    dg = dg_ref[0] + dg_ref[1]
    og_ref[...] = lax.rsqrt(dg[:N_GENES, 0:1] + 1.0)


def _mm_body(x_ref, w_ref, o_ref):
    o_ref[...] = jnp.dot(x_ref[...], w_ref[...],
                         preferred_element_type=jnp.float32)


def _scale_body(h1_ref, dc_ref, h3_ref, dg_ref, g1_ref, g3_ref):
    g1_ref[...] = h1_ref[...] * dc_ref[...]
    g3_ref[...] = h3_ref[...] * dg_ref[...]


def _gn_narrow_body(ac_ref, g1_ref, dc_ref, b1_ref, w1_ref, bb1_ref, a1_ref,
                    ag_ref, g3_ref, dg_ref, b3_ref, w3_ref, bb3_ref, a3_ref,
                    g2_ref, g4_ref):
    def one(ac, g, dis, b, w, bb, a):
        n = g.shape[0]
        t = dis * (ac[0, :n] + ac[1, :n] + g) + b
        m = jnp.mean(t, axis=0, keepdims=True)
        c = t - a * m
        v = jnp.mean(c * c, axis=0, keepdims=True)
        h = jnp.maximum(c * lax.rsqrt(v + _EPS) * w + bb, 0.0)
        return dis * h

    g2_ref[...] = one(ac_ref[...], g1_ref[...], dc_ref[...], b1_ref[...],
                      w1_ref[...], bb1_ref[...], a1_ref[...])
    g4_ref[...] = one(ag_ref[...], g3_ref[...], dg_ref[...], b3_ref[...],
                      w3_ref[...], bb3_ref[...], a3_ref[...])


def _agg_body(ac_ref, g2_ref, dc_ref, ag_ref, g4_ref, dg_ref,
              aggc_ref, aggg_ref):
    aggc_ref[...] = dc_ref[...] * (ac_ref[0, :N_CELLS] + ac_ref[1, :N_CELLS]
                                   + g2_ref[...])
    aggg_ref[...] = dg_ref[...] * (ag_ref[0, :N_GENES] + ag_ref[1, :N_GENES]
                                   + g4_ref[...])


def _l2c_body(agg_ref, w_ref, b_ref, ga_ref, gw_ref, gb_ref, o_ref):
    y = jnp.dot(agg_ref[...], w_ref[...],
                preferred_element_type=jnp.float32) + b_ref[...]
    m = jnp.mean(y, axis=0, keepdims=True)
    c = y - ga_ref[...] * m
    v = jnp.mean(c * c, axis=0, keepdims=True)
    o_ref[...] = jnp.maximum(c * lax.rsqrt(v + _EPS) * gw_ref[...]
                             + gb_ref[...], 0.0)


def _l2g_body(w4_ref, agg_ref, b4_ref, a4_ref, gw4_ref, gb4_ref, h_ref,
              o_ref):
    yt = lax.dot_general(w4_ref[...], agg_ref[...], (((0,), (1,)), ((), ())),
                         preferred_element_type=jnp.float32) + b4_ref[...]
    m = jnp.mean(yt, axis=1, keepdims=True)
    c = yt - a4_ref[...] * m
    v = jnp.mean(c * c, axis=1, keepdims=True)
    htt = jnp.maximum(c * lax.rsqrt(v + _EPS) * gw4_ref[...] + gb4_ref[...],
                      0.0)
    o_ref[...] = h_ref[...] + htt


def kernel(data, x, adj, x_t, adj_t, clustering, W1, b1, W2, b2, W3, b3, W4,
           b4, gn1_w, gn1_b, gn1_a, gn2_w, gn2_b, gn2_a, gn3_w, gn3_b, gn3_a,
           gn4_w, gn4_b, gn4_a, bn1_g, bn1_b, bn2_g, bn2_b):
    f32 = jnp.float32
    srcc, dstc = _prep_edges(adj, N_CELLS, KC)
    srcg, dstg = _prep_edges(adj_t, N_GENES, KG)
    zc_rows = jnp.zeros((ZR_C, H), f32)
    zg_rows = jnp.zeros((ZR_G, H), f32)
    ones_rows = jnp.ones((CHUNK, H), f32)

    degc_p, degg_p = _sc_hist(dstc, dstg, ones_rows, zc_rows, zg_rows)

    disC, disG = pl.pallas_call(
        _dis_body,
        out_shape=(jax.ShapeDtypeStruct((N_CELLS, 1), f32),
                   jax.ShapeDtypeStruct((N_GENES, 1), f32)),
    )(degc_p, degg_p)

    h1 = pl.pallas_call(
        _mm_body,
        grid=(10,),
        in_specs=[pl.BlockSpec((N_CELLS // 10, N_GENES), lambda i: (i, 0)),
                  pl.BlockSpec((N_GENES, H), lambda i: (0, 0))],
        out_specs=pl.BlockSpec((N_CELLS // 10, H), lambda i: (i, 0)),
        out_shape=jax.ShapeDtypeStruct((N_CELLS, H), f32),
    )(x, W1)
    h3 = pl.pallas_call(
        _mm_body,
        grid=(10,),
        in_specs=[pl.BlockSpec((N_GENES // 10, N_CELLS), lambda i: (i, 0)),
                  pl.BlockSpec((N_CELLS, H), lambda i: (0, 0))],
        out_specs=pl.BlockSpec((N_GENES // 10, H), lambda i: (i, 0)),
        out_shape=jax.ShapeDtypeStruct((N_GENES, H), f32),
    )(x_t, W3)

    g1, g3 = pl.pallas_call(
        _scale_body,
        out_shape=(jax.ShapeDtypeStruct((N_CELLS, H), f32),
                   jax.ShapeDtypeStruct((N_GENES, H), f32)),
    )(h1, disC, h3, disG)

    acc1_p = _sc_scatter_c(g1, srcc, dstc, zc_rows)
    acc3_p = _sc_scatter_g(g3, srcg, dstg, zg_rows)

    rb1 = b1.reshape(1, H)
    rw1, rbb1, ra1 = (gn1_w.reshape(1, H), gn1_b.reshape(1, H),
                      gn1_a.reshape(1, H))
    rb3 = b3.reshape(1, H)
    rw3, rbb3, ra3 = (gn3_w.reshape(1, H), gn3_b.reshape(1, H),
                      gn3_a.reshape(1, H))
    g2, g4 = pl.pallas_call(
        _gn_narrow_body,
        out_shape=(jax.ShapeDtypeStruct((N_CELLS, H), f32),
                   jax.ShapeDtypeStruct((N_GENES, H), f32)),
    )(acc1_p, g1, disC, rb1, rw1, rbb1, ra1,
      acc3_p, g3, disG, rb3, rw3, rbb3, ra3)

    acc2_p = _sc_scatter_c(g2, srcc, dstc, zc_rows)
    acc4_p = _sc_scatter_g(g4, srcg, dstg, zg_rows)

    agg2, agg4 = pl.pallas_call(
        _agg_body,
        out_shape=(jax.ShapeDtypeStruct((N_CELLS, H), f32),
                   jax.ShapeDtypeStruct((N_GENES, H), f32)),
    )(acc2_p, g2, disC, acc4_p, g4, disG)

    CB = 256
    h = pl.pallas_call(
        _l2c_body,
        grid=(pl.cdiv(N_GENES, CB),),
        in_specs=[pl.BlockSpec((N_CELLS, H), lambda i: (0, 0)),
                  pl.BlockSpec((H, CB), lambda i: (0, i)),
                  pl.BlockSpec((1, CB), lambda i: (0, i)),
                  pl.BlockSpec((1, CB), lambda i: (0, i)),
                  pl.BlockSpec((1, CB), lambda i: (0, i)),
                  pl.BlockSpec((1, CB), lambda i: (0, i))],
        out_specs=pl.BlockSpec((N_CELLS, CB), lambda i: (0, i)),
        out_shape=jax.ShapeDtypeStruct((N_CELLS, N_GENES), f32),
    )(agg2, W2, b2.reshape(1, N_GENES), gn2_a.reshape(1, N_GENES),
      gn2_w.reshape(1, N_GENES), gn2_b.reshape(1, N_GENES))

    RB = 512
    cb4 = b4.reshape(N_CELLS, 1)
    ca4, cw4, cbb4 = (gn4_a.reshape(N_CELLS, 1), gn4_w.reshape(N_CELLS, 1),
                      gn4_b.reshape(N_CELLS, 1))
    res = pl.pallas_call(
        _l2g_body,
        grid=(pl.cdiv(N_CELLS, RB),),
        in_specs=[pl.BlockSpec((H, RB), lambda i: (0, i)),
                  pl.BlockSpec((N_GENES, H), lambda i: (0, 0)),
                  pl.BlockSpec((RB, 1), lambda i: (i, 0)),
                  pl.BlockSpec((RB, 1), lambda i: (i, 0)),
                  pl.BlockSpec((RB, 1), lambda i: (i, 0)),
                  pl.BlockSpec((RB, 1), lambda i: (i, 0)),
                  pl.BlockSpec((RB, N_GENES), lambda i: (i, 0))],
        out_specs=pl.BlockSpec((RB, N_GENES), lambda i: (i, 0)),
        out_shape=jax.ShapeDtypeStruct((N_CELLS, N_GENES), f32),
    )(W4, agg4, cb4, ca4, cw4, cbb4, h)

    def _clustered():
        def bn(v, g, beta):
            m = jnp.mean(v, axis=0, keepdims=True)
            var = jnp.var(v, axis=0, keepdims=True)
            return (v - m) / jnp.sqrt(var + _EPS) * g + beta

        return bn(data, bn1_g, bn1_b) + bn(data.T, bn2_g, bn2_b).T

    return lax.cond(clustering != 0, _clustered, lambda: res)


# speedup vs baseline: 12.9874x; 1.8783x over previous
import functools

import jax
import jax.numpy as jnp
from jax import lax
from jax.experimental import pallas as pl
from jax.experimental.pallas import tpu as pltpu
from jax.experimental.pallas import tpu_sc as plsc

N_CELLS = 10000
N_GENES = 2000
H = 128
E_CELL = 320000
E_GENE = 64000

NC = 2
NS = 16
NW = NC * NS
CHUNK = 128

KC = E_CELL // (NW * CHUNK) + (1 if E_CELL % (NW * CHUNK) else 0)
KC += KC % 2
KG = E_GENE // (NW * CHUNK) + (1 if E_GENE % (NW * CHUNK) else 0)

ZR_C = 632
ZR_G = 128
ACC_C = NS * ZR_C
ACC_G = NS * ZR_G

_EPS = 1e-5

_SC_MESH = plsc.VectorSubcoreMesh(core_axis_name="c", subcore_axis_name="s")


def _prep_edges(adj, n_nodes, k_chunks, spare):
    e = adj.shape[0]
    pad = NW * k_chunks * CHUNK - e
    pidx = jnp.arange(pad, dtype=jnp.int32)
    src = jnp.concatenate([adj[:, 0], pidx % n_nodes])
    dst = jnp.concatenate([adj[:, 1], n_nodes + pidx % spare])
    return (src.reshape(NW, k_chunks, CHUNK), dst.reshape(NW, k_chunks, CHUNK))


@functools.partial(
    pl.kernel,
    out_type=(
        jax.ShapeDtypeStruct((NC, ACC_C, H), jnp.float32),
        jax.ShapeDtypeStruct((NC, ACC_G, H), jnp.float32),
    ),
    mesh=_SC_MESH,
    scratch_types=[
        pltpu.VMEM((KC, CHUNK), jnp.int32),
        pltpu.VMEM((KG, CHUNK), jnp.int32),
        pltpu.VMEM((CHUNK, H), jnp.float32),
        pltpu.VMEM_SHARED((ACC_C, H), jnp.float32),
        pltpu.VMEM_SHARED((ACC_G, H), jnp.float32),
    ],
)
def _sc_hist(dstc_hbm, dstg_hbm, ones_hbm, zc_hbm, zg_hbm, outc_hbm, outg_hbm,
             dstc_v, dstg_v, ones_v, degc_sh, degg_sh):
    cid = lax.axis_index("c")
    sid = lax.axis_index("s")
    wid = cid * NS + sid
    pltpu.sync_copy(dstc_hbm.at[wid], dstc_v)
    pltpu.sync_copy(dstg_hbm.at[wid], dstg_v)
    pltpu.sync_copy(ones_hbm, ones_v)
    pltpu.sync_copy(zc_hbm, degc_sh.at[pl.ds(sid * ZR_C, ZR_C)])
    pltpu.sync_copy(zg_hbm, degg_sh.at[pl.ds(sid * ZR_G, ZR_G)])
    plsc.subcore_barrier()

    def cbody(j, carry):
        pltpu.sync_copy(ones_v, degc_sh.at[dstc_v.at[j]], add=True)
        return carry

    lax.fori_loop(0, KC, cbody, 0)

    def gbody(j, carry):
        pltpu.sync_copy(ones_v, degg_sh.at[dstg_v.at[j]], add=True)
        return carry

    lax.fori_loop(0, KG, gbody, 0)
    plsc.subcore_barrier()
    pltpu.sync_copy(degc_sh.at[pl.ds(sid * ZR_C, ZR_C)],
                    outc_hbm.at[cid, pl.ds(sid * ZR_C, ZR_C)])
    pltpu.sync_copy(degg_sh.at[pl.ds(sid * ZR_G, ZR_G)],
                    outg_hbm.at[cid, pl.ds(sid * ZR_G, ZR_G)])


def _make_scatter(acc_rows, zr, k_chunks):
    @functools.partial(
        pl.kernel,
        out_type=jax.ShapeDtypeStruct((NC, acc_rows, H), jnp.float32),
        mesh=_SC_MESH,
        scratch_types=[
            pltpu.VMEM((k_chunks, CHUNK), jnp.int32),
            pltpu.VMEM((k_chunks, CHUNK), jnp.int32),
            pltpu.VMEM((CHUNK, H), jnp.float32),
            pltpu.VMEM_SHARED((acc_rows, H), jnp.float32),
        ],
    )
    def _scatter(g_hbm, src_hbm, dst_hbm, z_hbm, out_hbm,
                 src_v, dst_v, rows_v, acc_sh):
        cid = lax.axis_index("c")
        sid = lax.axis_index("s")
        wid = cid * NS + sid
        pltpu.sync_copy(src_hbm.at[wid], src_v)
        pltpu.sync_copy(dst_hbm.at[wid], dst_v)
        pltpu.sync_copy(z_hbm, acc_sh.at[pl.ds(sid * zr, zr)])
        plsc.subcore_barrier()

        def body(j, carry):
            pltpu.sync_copy(g_hbm.at[src_v.at[j]], rows_v)
            pltpu.sync_copy(rows_v, acc_sh.at[dst_v.at[j]], add=True)
            return carry

        lax.fori_loop(0, k_chunks, body, 0)
        plsc.subcore_barrier()
        pltpu.sync_copy(acc_sh.at[pl.ds(sid * zr, zr)],
                        out_hbm.at[cid, pl.ds(sid * zr, zr)])

    return _scatter


_sc_scatter_c = _make_scatter(ACC_C, ZR_C, KC)
_sc_scatter_g = _make_scatter(ACC_G, ZR_G, KG)


def _dis_body(dc_ref, dg_ref, oc_ref, og_ref):
    dc = dc_ref[0] + dc_ref[1]
    oc_ref[...] = lax.rsqrt(dc[:N_CELLS, 0:1] + 1.0)
    dg = dg_ref[0] + dg_ref[1]
    og_ref[...] = lax.rsqrt(dg[:N_GENES, 0:1] + 1.0)


def _mm_body(x_ref, w_ref, o_ref):
    o_ref[...] = jnp.dot(x_ref[...], w_ref[...],
                         preferred_element_type=jnp.float32)


def _scale_body(h1_ref, dc_ref, h3_ref, dg_ref, g1_ref, g3_ref):
    g1_ref[...] = h1_ref[...] * dc_ref[...]
    g3_ref[...] = h3_ref[...] * dg_ref[...]


def _gn_narrow_body(ac_ref, g1_ref, dc_ref, b1_ref, w1_ref, bb1_ref, a1_ref,
                    ag_ref, g3_ref, dg_ref, b3_ref, w3_ref, bb3_ref, a3_ref,
                    g2_ref, g4_ref):
    def one(ac, g, dis, b, w, bb, a):
        n = g.shape[0]
        t = dis * (ac[0, :n] + ac[1, :n] + g) + b
        m = jnp.mean(t, axis=0, keepdims=True)
        c = t - a * m
        v = jnp.mean(c * c, axis=0, keepdims=True)
        h = jnp.maximum(c * lax.rsqrt(v + _EPS) * w + bb, 0.0)
        return dis * h

    g2_ref[...] = one(ac_ref[...], g1_ref[...], dc_ref[...], b1_ref[...],
                      w1_ref[...], bb1_ref[...], a1_ref[...])
    g4_ref[...] = one(ag_ref[...], g3_ref[...], dg_ref[...], b3_ref[...],
                      w3_ref[...], bb3_ref[...], a3_ref[...])


def _agg_body(ac_ref, g2_ref, dc_ref, ag_ref, g4_ref, dg_ref,
              aggc_ref, aggg_ref):
    aggc_ref[...] = dc_ref[...] * (ac_ref[0, :N_CELLS] + ac_ref[1, :N_CELLS]
                                   + g2_ref[...])
    aggg_ref[...] = dg_ref[...] * (ag_ref[0, :N_GENES] + ag_ref[1, :N_GENES]
                                   + g4_ref[...])


def _l2c_body(agg_ref, w_ref, b_ref, ga_ref, gw_ref, gb_ref, o_ref):
    y = jnp.dot(agg_ref[...], w_ref[...],
                preferred_element_type=jnp.float32) + b_ref[...]
    m = jnp.mean(y, axis=0, keepdims=True)
    c = y - ga_ref[...] * m
    v = jnp.mean(c * c, axis=0, keepdims=True)
    o_ref[...] = jnp.maximum(c * lax.rsqrt(v + _EPS) * gw_ref[...]
                             + gb_ref[...], 0.0)


def _l2g_body(w4_ref, agg_ref, b4_ref, a4_ref, gw4_ref, gb4_ref, h_ref,
              o_ref):
    yt = lax.dot_general(w4_ref[...], agg_ref[...], (((0,), (1,)), ((), ())),
                         preferred_element_type=jnp.float32) + b4_ref[...]
    m = jnp.mean(yt, axis=1, keepdims=True)
    c = yt - a4_ref[...] * m
    v = jnp.mean(c * c, axis=1, keepdims=True)
    htt = jnp.maximum(c * lax.rsqrt(v + _EPS) * gw4_ref[...] + gb4_ref[...],
                      0.0)
    o_ref[...] = h_ref[...] + htt


def kernel(data, x, adj, x_t, adj_t, clustering, W1, b1, W2, b2, W3, b3, W4,
           b4, gn1_w, gn1_b, gn1_a, gn2_w, gn2_b, gn2_a, gn3_w, gn3_b, gn3_a,
           gn4_w, gn4_b, gn4_a, bn1_g, bn1_b, bn2_g, bn2_b):
    f32 = jnp.float32
    srcc, dstc = _prep_edges(adj, N_CELLS, KC, ACC_C - N_CELLS)
    srcg, dstg = _prep_edges(adj_t, N_GENES, KG, ACC_G - N_GENES)
    zc_rows = jnp.zeros((ZR_C, H), f32)
    zg_rows = jnp.zeros((ZR_G, H), f32)
    ones_rows = jnp.ones((CHUNK, H), f32)

    degc_p, degg_p = _sc_hist(dstc, dstg, ones_rows, zc_rows, zg_rows)

    disC, disG = pl.pallas_call(
        _dis_body,
        out_shape=(jax.ShapeDtypeStruct((N_CELLS, 1), f32),
                   jax.ShapeDtypeStruct((N_GENES, 1), f32)),
    )(degc_p, degg_p)

    h1 = pl.pallas_call(
        _mm_body,
        grid=(10,),
        in_specs=[pl.BlockSpec((N_CELLS // 10, N_GENES), lambda i: (i, 0)),
                  pl.BlockSpec((N_GENES, H), lambda i: (0, 0))],
        out_specs=pl.BlockSpec((N_CELLS // 10, H), lambda i: (i, 0)),
        out_shape=jax.ShapeDtypeStruct((N_CELLS, H), f32),
    )(x, W1)
    h3 = pl.pallas_call(
        _mm_body,
        grid=(10,),
        in_specs=[pl.BlockSpec((N_GENES // 10, N_CELLS), lambda i: (i, 0)),
                  pl.BlockSpec((N_CELLS, H), lambda i: (0, 0))],
        out_specs=pl.BlockSpec((N_GENES // 10, H), lambda i: (i, 0)),
        out_shape=jax.ShapeDtypeStruct((N_GENES, H), f32),
    )(x_t, W3)

    g1, g3 = pl.pallas_call(
        _scale_body,
        out_shape=(jax.ShapeDtypeStruct((N_CELLS, H), f32),
                   jax.ShapeDtypeStruct((N_GENES, H), f32)),
    )(h1, disC, h3, disG)

    acc1_p = _sc_scatter_c(g1, srcc, dstc, zc_rows)
    acc3_p = _sc_scatter_g(g3, srcg, dstg, zg_rows)

    rb1 = b1.reshape(1, H)
    rw1, rbb1, ra1 = (gn1_w.reshape(1, H), gn1_b.reshape(1, H),
                      gn1_a.reshape(1, H))
    rb3 = b3.reshape(1, H)
    rw3, rbb3, ra3 = (gn3_w.reshape(1, H), gn3_b.reshape(1, H),
                      gn3_a.reshape(1, H))
    g2, g4 = pl.pallas_call(
        _gn_narrow_body,
        out_shape=(jax.ShapeDtypeStruct((N_CELLS, H), f32),
                   jax.ShapeDtypeStruct((N_GENES, H), f32)),
    )(acc1_p, g1, disC, rb1, rw1, rbb1, ra1,
      acc3_p, g3, disG, rb3, rw3, rbb3, ra3)

    acc2_p = _sc_scatter_c(g2, srcc, dstc, zc_rows)
    acc4_p = _sc_scatter_g(g4, srcg, dstg, zg_rows)

    agg2, agg4 = pl.pallas_call(
        _agg_body,
        out_shape=(jax.ShapeDtypeStruct((N_CELLS, H), f32),
                   jax.ShapeDtypeStruct((N_GENES, H), f32)),
    )(acc2_p, g2, disC, acc4_p, g4, disG)

    CB = 256
    h = pl.pallas_call(
        _l2c_body,
        grid=(pl.cdiv(N_GENES, CB),),
        in_specs=[pl.BlockSpec((N_CELLS, H), lambda i: (0, 0)),
                  pl.BlockSpec((H, CB), lambda i: (0, i)),
                  pl.BlockSpec((1, CB), lambda i: (0, i)),
                  pl.BlockSpec((1, CB), lambda i: (0, i)),
                  pl.BlockSpec((1, CB), lambda i: (0, i)),
                  pl.BlockSpec((1, CB), lambda i: (0, i))],
        out_specs=pl.BlockSpec((N_CELLS, CB), lambda i: (0, i)),
        out_shape=jax.ShapeDtypeStruct((N_CELLS, N_GENES), f32),
    )(agg2, W2, b2.reshape(1, N_GENES), gn2_a.reshape(1, N_GENES),
      gn2_w.reshape(1, N_GENES), gn2_b.reshape(1, N_GENES))

    RB = 512
    cb4 = b4.reshape(N_CELLS, 1)
    ca4, cw4, cbb4 = (gn4_a.reshape(N_CELLS, 1), gn4_w.reshape(N_CELLS, 1),
                      gn4_b.reshape(N_CELLS, 1))
    res = pl.pallas_call(
        _l2g_body,
        grid=(pl.cdiv(N_CELLS, RB),),
        in_specs=[pl.BlockSpec((H, RB), lambda i: (0, i)),
                  pl.BlockSpec((N_GENES, H), lambda i: (0, 0)),
                  pl.BlockSpec((RB, 1), lambda i: (i, 0)),
                  pl.BlockSpec((RB, 1), lambda i: (i, 0)),
                  pl.BlockSpec((RB, 1), lambda i: (i, 0)),
                  pl.BlockSpec((RB, 1), lambda i: (i, 0)),
                  pl.BlockSpec((RB, N_GENES), lambda i: (i, 0))],
        out_specs=pl.BlockSpec((RB, N_GENES), lambda i: (i, 0)),
        out_shape=jax.ShapeDtypeStruct((N_CELLS, N_GENES), f32),
    )(W4, agg4, cb4, ca4, cw4, cbb4, h)

    def _clustered():
        def bn(v, g, beta):
            m = jnp.mean(v, axis=0, keepdims=True)
            var = jnp.var(v, axis=0, keepdims=True)
            return (v - m) / jnp.sqrt(var + _EPS) * g + beta

        return bn(data, bn1_g, bn1_b) + bn(data.T, bn2_g, bn2_b).T

    return lax.cond(clustering != 0, _clustered, lambda: res)


# speedup vs baseline: 13.8404x; 1.0657x over previous
import functools

import jax
import jax.numpy as jnp
from jax import lax
from jax.experimental import pallas as pl
from jax.experimental.pallas import tpu as pltpu
from jax.experimental.pallas import tpu_sc as plsc

N_CELLS = 10000
N_GENES = 2000
H = 128
E_CELL = 320000
E_GENE = 64000

NC = 2
NS = 16
NW = NC * NS
CHUNK = 128

KC = E_CELL // (NW * CHUNK) + (1 if E_CELL % (NW * CHUNK) else 0)
KC += KC % 2
KG = E_GENE // (NW * CHUNK) + (1 if E_GENE % (NW * CHUNK) else 0)

ZR_C = 632
ZR_G = 128
ACC_C = NS * ZR_C
ACC_G = NS * ZR_G

_EPS = 1e-5

_SC_MESH = plsc.VectorSubcoreMesh(core_axis_name="c", subcore_axis_name="s")


def _prep_edges(adj, n_nodes, k_chunks, spare):
    e = adj.shape[0]
    pad = NW * k_chunks * CHUNK - e
    pidx = jnp.arange(pad, dtype=jnp.int32)
    src = jnp.concatenate([adj[:, 0], pidx % n_nodes])
    dst = jnp.concatenate([adj[:, 1], n_nodes + pidx % spare])
    return (src.reshape(NW, k_chunks, CHUNK), dst.reshape(NW, k_chunks, CHUNK))


@functools.partial(
    pl.kernel,
    out_type=(
        jax.ShapeDtypeStruct((NC, ACC_C, H), jnp.float32),
        jax.ShapeDtypeStruct((NC, ACC_G, H), jnp.float32),
    ),
    mesh=_SC_MESH,
    scratch_types=[
        pltpu.VMEM((KC, CHUNK), jnp.int32),
        pltpu.VMEM((KG, CHUNK), jnp.int32),
        pltpu.VMEM((CHUNK, H), jnp.float32),
        pltpu.VMEM_SHARED((ACC_C, H), jnp.float32),
        pltpu.VMEM_SHARED((ACC_G, H), jnp.float32),
    ],
)
def _sc_hist(dstc_hbm, dstg_hbm, ones_hbm, zc_hbm, zg_hbm, outc_hbm, outg_hbm,
             dstc_v, dstg_v, ones_v, degc_sh, degg_sh):
    cid = lax.axis_index("c")
    sid = lax.axis_index("s")
    wid = cid * NS + sid
    pltpu.sync_copy(dstc_hbm.at[wid], dstc_v)
    pltpu.sync_copy(dstg_hbm.at[wid], dstg_v)
    pltpu.sync_copy(ones_hbm, ones_v)
    pltpu.sync_copy(zc_hbm, degc_sh.at[pl.ds(sid * ZR_C, ZR_C)])
    pltpu.sync_copy(zg_hbm, degg_sh.at[pl.ds(sid * ZR_G, ZR_G)])
    plsc.subcore_barrier()

    def cbody(j, carry):
        pltpu.sync_copy(ones_v, degc_sh.at[dstc_v.at[j]], add=True)
        return carry

    lax.fori_loop(0, KC, cbody, 0)

    def gbody(j, carry):
        pltpu.sync_copy(ones_v, degg_sh.at[dstg_v.at[j]], add=True)
        return carry

    lax.fori_loop(0, KG, gbody, 0)
    plsc.subcore_barrier()
    pltpu.sync_copy(degc_sh.at[pl.ds(sid * ZR_C, ZR_C)],
                    outc_hbm.at[cid, pl.ds(sid * ZR_C, ZR_C)])
    pltpu.sync_copy(degg_sh.at[pl.ds(sid * ZR_G, ZR_G)],
                    outg_hbm.at[cid, pl.ds(sid * ZR_G, ZR_G)])


def _make_scatter(acc_rows, zr, k_chunks):
    @functools.partial(
        pl.kernel,
        out_type=jax.ShapeDtypeStruct((NC, acc_rows, H), jnp.float32),
        mesh=_SC_MESH,
        scratch_types=[
            pltpu.VMEM((k_chunks, CHUNK), jnp.int32),
            pltpu.VMEM((k_chunks, CHUNK), jnp.int32),
            pltpu.VMEM((CHUNK, H), jnp.float32),
            pltpu.VMEM_SHARED((acc_rows, H), jnp.float32),
        ],
    )
    def _scatter(g_hbm, src_hbm, dst_hbm, z_hbm, out_hbm,
                 src_v, dst_v, rows_v, acc_sh):
        cid = lax.axis_index("c")
        sid = lax.axis_index("s")
        wid = cid * NS + sid
        pltpu.sync_copy(src_hbm.at[wid], src_v)
        pltpu.sync_copy(dst_hbm.at[wid], dst_v)
        pltpu.sync_copy(z_hbm, acc_sh.at[pl.ds(sid * zr, zr)])
        plsc.subcore_barrier()

        def body(j, carry):
            pltpu.sync_copy(g_hbm.at[src_v.at[j]], rows_v)
            pltpu.sync_copy(rows_v, acc_sh.at[dst_v.at[j]], add=True)
            return carry

        lax.fori_loop(0, k_chunks, body, 0)
        plsc.subcore_barrier()
        pltpu.sync_copy(acc_sh.at[pl.ds(sid * zr, zr)],
                        out_hbm.at[cid, pl.ds(sid * zr, zr)])

    return _scatter


_sc_scatter_c = _make_scatter(ACC_C, ZR_C, KC)
_sc_scatter_g = _make_scatter(ACC_G, ZR_G, KG)


def _dis_body(dc_ref, dg_ref, oc_ref, og_ref):
    dc = dc_ref[0] + dc_ref[1]
    oc_ref[...] = lax.rsqrt(dc[:N_CELLS, 0:1] + 1.0)
    dg = dg_ref[0] + dg_ref[1]
    og_ref[...] = lax.rsqrt(dg[:N_GENES, 0:1] + 1.0)


def _mm_body(x_ref, w_ref, o_ref):
    o_ref[...] = jnp.dot(x_ref[...], w_ref[...],
                         preferred_element_type=jnp.float32)


def _scale_body(h1_ref, dc_ref, h3_ref, dg_ref, g1_ref, g3_ref):
    g1_ref[...] = h1_ref[...] * dc_ref[...]
    g3_ref[...] = h3_ref[...] * dg_ref[...]


def _gn_narrow_body(ac_ref, g1_ref, dc_ref, b1_ref, w1_ref, bb1_ref, a1_ref,
                    ag_ref, g3_ref, dg_ref, b3_ref, w3_ref, bb3_ref, a3_ref,
                    g2_ref, g4_ref):
    def one(ac, g, dis, b, w, bb, a):
        n = g.shape[0]
        t = dis * (ac[0, :n] + ac[1, :n] + g) + b
        m = jnp.mean(t, axis=0, keepdims=True)
        c = t - a * m
        v = jnp.mean(c * c, axis=0, keepdims=True)
        h = jnp.maximum(c * lax.rsqrt(v + _EPS) * w + bb, 0.0)
        return dis * h

    g2_ref[...] = one(ac_ref[...], g1_ref[...], dc_ref[...], b1_ref[...],
                      w1_ref[...], bb1_ref[...], a1_ref[...])
    g4_ref[...] = one(ag_ref[...], g3_ref[...], dg_ref[...], b3_ref[...],
                      w3_ref[...], bb3_ref[...], a3_ref[...])


def _agg_body(ac_ref, g2_ref, dc_ref, ag_ref, g4_ref, dg_ref,
              aggc_ref, aggg_ref):
    aggc_ref[...] = dc_ref[...] * (ac_ref[0, :N_CELLS] + ac_ref[1, :N_CELLS]
                                   + g2_ref[...])
    aggg_ref[...] = dg_ref[...] * (ag_ref[0, :N_GENES] + ag_ref[1, :N_GENES]
                                   + g4_ref[...])


def _l2c_stats_body(agg_ref, w_ref, b_ref, s1_ref, s2_ref):
    i = pl.program_id(0)
    y = jnp.dot(agg_ref[...], w_ref[...],
                preferred_element_type=jnp.float32) + b_ref[...]
    s1 = jnp.sum(y, axis=0, keepdims=True)
    s2 = jnp.sum(y * y, axis=0, keepdims=True)

    @pl.when(i == 0)
    def _():
        s1_ref[...] = s1
        s2_ref[...] = s2

    @pl.when(i > 0)
    def _():
        s1_ref[...] += s1
        s2_ref[...] += s2


def _l2_fused_body(w4_ref, agg4_ref, b4_ref, a4_ref, gw4_ref, gb4_ref,
                   agg2_ref, w2_ref, b2_ref, s1_ref, s2_ref, a2_ref,
                   gw2_ref, gb2_ref, o_ref):
    y = jnp.dot(agg2_ref[...], w2_ref[...],
                preferred_element_type=jnp.float32) + b2_ref[...]
    m = s1_ref[...] * (1.0 / N_CELLS)
    ey2 = s2_ref[...] * (1.0 / N_CELLS)
    am = a2_ref[...] * m
    v = ey2 - 2.0 * am * m + am * am
    h = jnp.maximum((y - am) * lax.rsqrt(v + _EPS) * gw2_ref[...]
                    + gb2_ref[...], 0.0)
    yt = lax.dot_general(w4_ref[...], agg4_ref[...], (((0,), (1,)), ((), ())),
                         preferred_element_type=jnp.float32) + b4_ref[...]
    mt = jnp.mean(yt, axis=1, keepdims=True)
    ct = yt - a4_ref[...] * mt
    vt = jnp.mean(ct * ct, axis=1, keepdims=True)
    htt = jnp.maximum(ct * lax.rsqrt(vt + _EPS) * gw4_ref[...] + gb4_ref[...],
                      0.0)
    o_ref[...] = h + htt


def kernel(data, x, adj, x_t, adj_t, clustering, W1, b1, W2, b2, W3, b3, W4,
           b4, gn1_w, gn1_b, gn1_a, gn2_w, gn2_b, gn2_a, gn3_w, gn3_b, gn3_a,
           gn4_w, gn4_b, gn4_a, bn1_g, bn1_b, bn2_g, bn2_b):
    f32 = jnp.float32
    srcc, dstc = _prep_edges(adj, N_CELLS, KC, ACC_C - N_CELLS)
    srcg, dstg = _prep_edges(adj_t, N_GENES, KG, ACC_G - N_GENES)
    zc_rows = jnp.zeros((ZR_C, H), f32)
    zg_rows = jnp.zeros((ZR_G, H), f32)
    ones_rows = jnp.ones((CHUNK, H), f32)

    degc_p, degg_p = _sc_hist(dstc, dstg, ones_rows, zc_rows, zg_rows)

    disC, disG = pl.pallas_call(
        _dis_body,
        out_shape=(jax.ShapeDtypeStruct((N_CELLS, 1), f32),
                   jax.ShapeDtypeStruct((N_GENES, 1), f32)),
    )(degc_p, degg_p)

    h1 = pl.pallas_call(
        _mm_body,
        grid=(10,),
        in_specs=[pl.BlockSpec((N_CELLS // 10, N_GENES), lambda i: (i, 0)),
                  pl.BlockSpec((N_GENES, H), lambda i: (0, 0))],
        out_specs=pl.BlockSpec((N_CELLS // 10, H), lambda i: (i, 0)),
        out_shape=jax.ShapeDtypeStruct((N_CELLS, H), f32),
    )(x, W1)
    h3 = pl.pallas_call(
        _mm_body,
        grid=(10,),
        in_specs=[pl.BlockSpec((N_GENES // 10, N_CELLS), lambda i: (i, 0)),
                  pl.BlockSpec((N_CELLS, H), lambda i: (0, 0))],
        out_specs=pl.BlockSpec((N_GENES // 10, H), lambda i: (i, 0)),
        out_shape=jax.ShapeDtypeStruct((N_GENES, H), f32),
    )(x_t, W3)

    g1, g3 = pl.pallas_call(
        _scale_body,
        out_shape=(jax.ShapeDtypeStruct((N_CELLS, H), f32),
                   jax.ShapeDtypeStruct((N_GENES, H), f32)),
    )(h1, disC, h3, disG)

    acc1_p = _sc_scatter_c(g1, srcc, dstc, zc_rows)
    acc3_p = _sc_scatter_g(g3, srcg, dstg, zg_rows)

    rb1 = b1.reshape(1, H)
    rw1, rbb1, ra1 = (gn1_w.reshape(1, H), gn1_b.reshape(1, H),
                      gn1_a.reshape(1, H))
    rb3 = b3.reshape(1, H)
    rw3, rbb3, ra3 = (gn3_w.reshape(1, H), gn3_b.reshape(1, H),
                      gn3_a.reshape(1, H))
    g2, g4 = pl.pallas_call(
        _gn_narrow_body,
        out_shape=(jax.ShapeDtypeStruct((N_CELLS, H), f32),
                   jax.ShapeDtypeStruct((N_GENES, H), f32)),
    )(acc1_p, g1, disC, rb1, rw1, rbb1, ra1,
      acc3_p, g3, disG, rb3, rw3, rbb3, ra3)

    acc2_p = _sc_scatter_c(g2, srcc, dstc, zc_rows)
    acc4_p = _sc_scatter_g(g4, srcg, dstg, zg_rows)

    agg2, agg4 = pl.pallas_call(
        _agg_body,
        out_shape=(jax.ShapeDtypeStruct((N_CELLS, H), f32),
                   jax.ShapeDtypeStruct((N_GENES, H), f32)),
    )(acc2_p, g2, disC, acc4_p, g4, disG)

    SB = 1000
    rb2, ra2 = b2.reshape(1, N_GENES), gn2_a.reshape(1, N_GENES)
    rw2, rbb2 = gn2_w.reshape(1, N_GENES), gn2_b.reshape(1, N_GENES)
    s1, s2 = pl.pallas_call(
        _l2c_stats_body,
        grid=(N_CELLS // SB,),
        in_specs=[pl.BlockSpec((SB, H), lambda i: (i, 0)),
                  pl.BlockSpec((H, N_GENES), lambda i: (0, 0)),
                  pl.BlockSpec((1, N_GENES), lambda i: (0, 0))],
        out_specs=(pl.BlockSpec((1, N_GENES), lambda i: (0, 0)),
                   pl.BlockSpec((1, N_GENES), lambda i: (0, 0))),
        out_shape=(jax.ShapeDtypeStruct((1, N_GENES), f32),
                   jax.ShapeDtypeStruct((1, N_GENES), f32)),
    )(agg2, W2, rb2)

    RB = 512
    cb4 = b4.reshape(N_CELLS, 1)
    ca4, cw4, cbb4 = (gn4_a.reshape(N_CELLS, 1), gn4_w.reshape(N_CELLS, 1),
                      gn4_b.reshape(N_CELLS, 1))
    res = pl.pallas_call(
        _l2_fused_body,
        grid=(pl.cdiv(N_CELLS, RB),),
        in_specs=[pl.BlockSpec((H, RB), lambda i: (0, i)),
                  pl.BlockSpec((N_GENES, H), lambda i: (0, 0)),
                  pl.BlockSpec((RB, 1), lambda i: (i, 0)),
                  pl.BlockSpec((RB, 1), lambda i: (i, 0)),
                  pl.BlockSpec((RB, 1), lambda i: (i, 0)),
                  pl.BlockSpec((RB, 1), lambda i: (i, 0)),
                  pl.BlockSpec((RB, H), lambda i: (i, 0)),
                  pl.BlockSpec((H, N_GENES), lambda i: (0, 0)),
                  pl.BlockSpec((1, N_GENES), lambda i: (0, 0)),
                  pl.BlockSpec((1, N_GENES), lambda i: (0, 0)),
                  pl.BlockSpec((1, N_GENES), lambda i: (0, 0)),
                  pl.BlockSpec((1, N_GENES), lambda i: (0, 0)),
                  pl.BlockSpec((1, N_GENES), lambda i: (0, 0)),
                  pl.BlockSpec((1, N_GENES), lambda i: (0, 0))],
        out_specs=pl.BlockSpec((RB, N_GENES), lambda i: (i, 0)),
        out_shape=jax.ShapeDtypeStruct((N_CELLS, N_GENES), f32),
    )(W4, agg4, cb4, ca4, cw4, cbb4,
      agg2, W2, rb2, s1, s2, ra2, rw2, rbb2)

    def _clustered():
        def bn(v, g, beta):
            m = jnp.mean(v, axis=0, keepdims=True)
            var = jnp.var(v, axis=0, keepdims=True)
            return (v - m) / jnp.sqrt(var + _EPS) * g + beta

        return bn(data, bn1_g, bn1_b) + bn(data.T, bn2_g, bn2_b).T

    return lax.cond(clustering != 0, _clustered, lambda: res)
